# baseline (speedup 1.0000x reference)
import jax
import jax.numpy as jnp
from jax.experimental import pallas as pl
from jax.experimental.pallas import tpu as pltpu

N_ = 64
P_ = 64
CSP = 128
CDN = 64
CO = 32
M_ = 32
BBLK = 8
DLANES = P_ * CDN

_PREC = jax.lax.Precision.HIGHEST


def _body(xt_ref, coor_ref, wsp_ref, wdn_ref, wc_ref,
          bsp_ref, ssp_ref, besp_ref,
          bdn_ref, sdn_ref, bedn_ref,
          bds_ref, sds_ref, beds_ref,
          spo_ref, outo_ref, coors_ref, sfps_scr):
    B = BBLK
    coor = coor_ref[...]
    lane_n = jax.lax.broadcasted_iota(jnp.int32, (B, N_), 1)

    def fps_step(t, carry):
        dists, far = carry
        onehot = (lane_n == far).astype(jnp.float32)
        sfps_scr[:, pl.ds(t, 1), :] = onehot[:, None, :]
        centroid = jnp.sum(coor * onehot[:, :, None], axis=1)
        coors_ref[:, pl.ds(t, 1), :] = centroid[:, None, :]
        d = jnp.sum((coor - centroid[:, None, :]) ** 2, axis=2)
        dists = jnp.minimum(dists, d)
        mx = jnp.max(dists, axis=1, keepdims=True)
        far = jnp.min(jnp.where(dists == mx, lane_n, N_), axis=1, keepdims=True)
        return dists, far

    carry0 = (jnp.full((B, N_), 1e10, jnp.float32),
              jnp.zeros((B, 1), jnp.int32))
    jax.lax.fori_loop(0, M_, fps_step, carry0)
    sfps = sfps_scr[...]
    centers = coors_ref[...]

    dc = jnp.sum((centers[:, :, None, :] - coor[:, None, :, :]) ** 2, axis=3)
    lane3 = jax.lax.broadcasted_iota(jnp.int32, (B, M_, N_), 2)
    mn0 = jnp.min(dc, axis=2, keepdims=True)
    i0 = jnp.min(jnp.where(dc == mn0, lane3, N_), axis=2, keepdims=True)
    s0 = lane3 == i0
    dc1 = jnp.where(s0, jnp.float32(jnp.inf), dc)
    mn1 = jnp.min(dc1, axis=2, keepdims=True)
    i1 = jnp.min(jnp.where(dc1 == mn1, lane3, N_), axis=2, keepdims=True)
    s1 = lane3 == i1
    sd0 = s0.astype(jnp.float32) - sfps
    sd1 = s1.astype(jnp.float32) - sfps

    wsp = wsp_ref[...]
    wdn = wdn_ref[...]
    wc = wc_ref[...]
    bsp = bsp_ref[...]; ssp = ssp_ref[...]; besp = besp_ref[...]
    bdn = bdn_ref[...]; sdn = sdn_ref[...]; bedn = bedn_ref[...]
    bds = bds_ref[...]; sds = sds_ref[...]; beds = beds_ref[...]

    def per_batch(b):
        scat = jnp.concatenate([sd0[b], sd1[b], sfps[b]], axis=0)
        g = jnp.dot(scat, xt_ref[b], precision=_PREC,
                    preferred_element_type=jnp.float32)
        d0 = g[0:32]; d1 = g[32:64]; cen = g[64:96]

        ysp0 = jnp.concatenate([d0[:, DLANES:], cen[:, DLANES:]], axis=1)
        ysp1 = jnp.concatenate([d1[:, DLANES:], cen[:, DLANES:]], axis=1)
        sp0 = jnp.dot(ysp0, wsp, precision=_PREC, preferred_element_type=jnp.float32)
        sp1 = jnp.dot(ysp1, wsp, precision=_PREC, preferred_element_type=jnp.float32)
        spm = (jnp.maximum(sp0, sp1) + bsp) * ssp + besp
        spo_ref[b] = jnp.where(spm > 0, spm, 0.2 * spm)

        half = M_ * P_ // 2
        y0 = d0[:, :DLANES].reshape(half, 2 * CDN)
        y1 = d1[:, :DLANES].reshape(half, 2 * CDN)
        yc = cen[:, :DLANES].reshape(half, 2 * CDN)
        o0 = jnp.dot(y0, wdn, precision=_PREC, preferred_element_type=jnp.float32)
        o1 = jnp.dot(y1, wdn, precision=_PREC, preferred_element_type=jnp.float32)
        oc = jnp.dot(yc, wdn, precision=_PREC, preferred_element_type=jnp.float32)
        first = jnp.maximum(o0, o1)
        ddf = (first + bdn) * sdn + bedn
        ddf = jnp.where(ddf > 0, ddf, 0.2 * ddf)
        ddc = (oc + bdn) * sdn + bedn
        ddc = jnp.where(ddc > 0, ddc, 0.2 * ddc)

        Q = P_ // 2
        ddf3 = ddf.reshape(M_, Q, CDN)
        ddc3 = ddc.reshape(M_, Q, CDN)
        even = jnp.concatenate([ddf3[:, :Q // 2, :], ddc3[:, :Q // 2, :]], axis=1)
        odd = jnp.concatenate([ddf3[:, Q // 2:, :], ddc3[:, Q // 2:, :]], axis=1)
        odd_sh = jnp.concatenate(
            [jnp.zeros((M_, 1, CDN), jnp.float32), odd[:, :Q - 1, :]], axis=1)
        co = (jnp.dot(even.reshape(M_ * Q, CDN), wc[CDN:2 * CDN], precision=_PREC,
                      preferred_element_type=jnp.float32)
              + jnp.dot(odd.reshape(M_ * Q, CDN), wc[2 * CDN:], precision=_PREC,
                        preferred_element_type=jnp.float32)
              + jnp.dot(odd_sh.reshape(M_ * Q, CDN), wc[:CDN], precision=_PREC,
                        preferred_element_type=jnp.float32))
        co = (co + bds) * sds + beds
        co = jnp.where(co > 0, co, 0.2 * co)
        outo_ref[b] = co.reshape(M_, P_ // 2, CDN)

    for b in range(B):
        per_batch(b)


def kernel(sparse_fea, dense_fea, stk_coor, n_stk_center,
           W_sp, b_sp, g_sp, be_sp,
           W_dn, b_dn, g_dn, be_dn,
           W_ds, b_ds, g_ds, be_ds):
    del n_stk_center
    bs = sparse_fea.shape[0]
    a_order = list(range(0, P_ // 2, 2)) + list(range(1, P_ // 2, 2))
    p_order = jnp.array([p for a in a_order for p in (2 * a, 2 * a + 1)],
                        dtype=jnp.int32)
    df_t = (jnp.transpose(dense_fea, (0, 2, 3, 1))[:, :, p_order, :]
            .reshape(bs, N_, DLANES))
    sf_t = jnp.swapaxes(sparse_fea, 1, 2)
    xt = jnp.concatenate([df_t, sf_t], axis=2)

    wsp_t = W_sp.T
    wdn_t = W_dn.T
    wc = jnp.transpose(W_ds[:, :, 0, :], (2, 1, 0)).reshape(3 * CDN, CDN)
    inv = jnp.float32(1.0) / jnp.sqrt(jnp.float32(1.0 + 1e-5))
    row = lambda v: v.reshape(1, v.shape[0])

    grid = (bs // BBLK,)
    blk = lambda shape: pl.BlockSpec(shape, lambda i: (i,) + (0,) * (len(shape) - 1))
    full = lambda shape: pl.BlockSpec(shape, lambda i: (0,) * len(shape))

    sp_pre, out_pre, coor_s = pl.pallas_call(
        _body,
        grid=grid,
        in_specs=[blk((BBLK, N_, DLANES + CSP)), blk((BBLK, N_, CO)),
                  full((2 * CSP, CSP)), full((2 * CDN, CDN)), full((3 * CDN, CDN)),
                  full((1, CSP)), full((1, CSP)), full((1, CSP)),
                  full((1, CDN)), full((1, CDN)), full((1, CDN)),
                  full((1, CDN)), full((1, CDN)), full((1, CDN))],
        out_specs=[blk((BBLK, M_, CSP)),
                   blk((BBLK, M_, P_ // 2, CDN)),
                   blk((BBLK, M_, CO))],
        out_shape=[jax.ShapeDtypeStruct((bs, M_, CSP), jnp.float32),
                   jax.ShapeDtypeStruct((bs, M_, P_ // 2, CDN), jnp.float32),
                   jax.ShapeDtypeStruct((bs, M_, CO), jnp.float32)],
        scratch_shapes=[pltpu.VMEM((BBLK, M_, N_), jnp.float32)],
    )(xt, stk_coor, wsp_t, wdn_t, wc,
      row(b_sp), row(g_sp * inv), row(be_sp),
      row(b_dn), row(g_dn * inv), row(be_dn),
      row(b_ds), row(g_ds * inv), row(be_ds))

    sparse_out = jnp.transpose(sp_pre, (0, 2, 1))
    out = jnp.transpose(out_pre, (0, 3, 1, 2))
    return (sparse_out, out, coor_s)


# speedup vs baseline: 1.6904x; 1.6904x over previous
import jax
import jax.numpy as jnp
from jax.experimental import pallas as pl
from jax.experimental.pallas import tpu as pltpu

N_ = 64
P_ = 64
CSP = 128
CDN = 64
CO = 32
M_ = 32
BBLK = 8
DLANES = P_ * CDN

_PREC = jax.lax.Precision.DEFAULT


def _body(xt_ref, coor_ref, wsp_ref, wdn_ref, wc_ref,
          bsp_ref, ssp_ref, besp_ref,
          bdn_ref, sdn_ref, bedn_ref,
          bds_ref, sds_ref, beds_ref,
          spo_ref, outo_ref, coors_ref, sfps_scr):
    B = BBLK
    coor = coor_ref[...]
    lane_n = jax.lax.broadcasted_iota(jnp.int32, (B, N_), 1)

    def fps_step(t, carry):
        dists, far = carry
        onehot = (lane_n == far).astype(jnp.float32)
        sfps_scr[:, pl.ds(t, 1), :] = onehot[:, None, :]
        centroid = jnp.sum(coor * onehot[:, :, None], axis=1)
        coors_ref[:, pl.ds(t, 1), :] = centroid[:, None, :]
        d = jnp.sum((coor - centroid[:, None, :]) ** 2, axis=2)
        dists = jnp.minimum(dists, d)
        mx = jnp.max(dists, axis=1, keepdims=True)
        far = jnp.min(jnp.where(dists == mx, lane_n, N_), axis=1, keepdims=True)
        return dists, far

    carry0 = (jnp.full((B, N_), 1e10, jnp.float32),
              jnp.zeros((B, 1), jnp.int32))
    jax.lax.fori_loop(0, M_, fps_step, carry0)
    sfps = sfps_scr[...]
    centers = coors_ref[...]

    dc = jnp.sum((centers[:, :, None, :] - coor[:, None, :, :]) ** 2, axis=3)
    lane3 = jax.lax.broadcasted_iota(jnp.int32, (B, M_, N_), 2)
    mn0 = jnp.min(dc, axis=2, keepdims=True)
    i0 = jnp.min(jnp.where(dc == mn0, lane3, N_), axis=2, keepdims=True)
    s0 = lane3 == i0
    dc1 = jnp.where(s0, jnp.float32(jnp.inf), dc)
    mn1 = jnp.min(dc1, axis=2, keepdims=True)
    i1 = jnp.min(jnp.where(dc1 == mn1, lane3, N_), axis=2, keepdims=True)
    s1 = lane3 == i1
    sd0 = s0.astype(jnp.float32) - sfps
    sd1 = s1.astype(jnp.float32) - sfps

    wsp = wsp_ref[...]
    wdn = wdn_ref[...]
    wc = wc_ref[...]
    bsp = bsp_ref[...]; ssp = ssp_ref[...]; besp = besp_ref[...]
    bdn = bdn_ref[...]; sdn = sdn_ref[...]; bedn = bedn_ref[...]
    bds = bds_ref[...]; sds = sds_ref[...]; beds = beds_ref[...]

    def per_batch(b):
        scat = jnp.concatenate([sd0[b], sd1[b], sfps[b]], axis=0)
        g = jnp.dot(scat, xt_ref[b], precision=_PREC,
                    preferred_element_type=jnp.float32)
        d0 = g[0:32]; d1 = g[32:64]; cen = g[64:96]

        ysp0 = jnp.concatenate([d0[:, DLANES:], cen[:, DLANES:]], axis=1)
        ysp1 = jnp.concatenate([d1[:, DLANES:], cen[:, DLANES:]], axis=1)
        sp0 = jnp.dot(ysp0, wsp, precision=_PREC, preferred_element_type=jnp.float32)
        sp1 = jnp.dot(ysp1, wsp, precision=_PREC, preferred_element_type=jnp.float32)
        spm = (jnp.maximum(sp0, sp1) + bsp) * ssp + besp
        spo_ref[b] = jnp.where(spm > 0, spm, 0.2 * spm)

        half = M_ * P_ // 2
        y0 = d0[:, :DLANES].reshape(half, 2 * CDN)
        y1 = d1[:, :DLANES].reshape(half, 2 * CDN)
        yc = cen[:, :DLANES].reshape(half, 2 * CDN)
        o0 = jnp.dot(y0, wdn, precision=_PREC, preferred_element_type=jnp.float32)
        o1 = jnp.dot(y1, wdn, precision=_PREC, preferred_element_type=jnp.float32)
        oc = jnp.dot(yc, wdn, precision=_PREC, preferred_element_type=jnp.float32)
        first = jnp.maximum(o0, o1)
        ddf = (first + bdn) * sdn + bedn
        ddf = jnp.where(ddf > 0, ddf, 0.2 * ddf)
        ddc = (oc + bdn) * sdn + bedn
        ddc = jnp.where(ddc > 0, ddc, 0.2 * ddc)

        Q = P_ // 2
        ddf3 = ddf.reshape(M_, Q, CDN)
        ddc3 = ddc.reshape(M_, Q, CDN)
        even = jnp.concatenate([ddf3[:, :Q // 2, :], ddc3[:, :Q // 2, :]], axis=1)
        odd = jnp.concatenate([ddf3[:, Q // 2:, :], ddc3[:, Q // 2:, :]], axis=1)
        odd_sh = jnp.concatenate(
            [jnp.zeros((M_, 1, CDN), jnp.float32), odd[:, :Q - 1, :]], axis=1)
        co = (jnp.dot(even.reshape(M_ * Q, CDN), wc[CDN:2 * CDN], precision=_PREC,
                      preferred_element_type=jnp.float32)
              + jnp.dot(odd.reshape(M_ * Q, CDN), wc[2 * CDN:], precision=_PREC,
                        preferred_element_type=jnp.float32)
              + jnp.dot(odd_sh.reshape(M_ * Q, CDN), wc[:CDN], precision=_PREC,
                        preferred_element_type=jnp.float32))
        co = (co + bds) * sds + beds
        co = jnp.where(co > 0, co, 0.2 * co)
        outo_ref[b] = co.reshape(M_, P_ // 2, CDN)

    for b in range(B):
        per_batch(b)


def kernel(sparse_fea, dense_fea, stk_coor, n_stk_center,
           W_sp, b_sp, g_sp, be_sp,
           W_dn, b_dn, g_dn, be_dn,
           W_ds, b_ds, g_ds, be_ds):
    del n_stk_center
    bs = sparse_fea.shape[0]
    a_order = list(range(0, P_ // 2, 2)) + list(range(1, P_ // 2, 2))
    p_order = jnp.array([p for a in a_order for p in (2 * a, 2 * a + 1)],
                        dtype=jnp.int32)
    df_t = (jnp.transpose(dense_fea, (0, 2, 3, 1))[:, :, p_order, :]
            .reshape(bs, N_, DLANES))
    sf_t = jnp.swapaxes(sparse_fea, 1, 2)
    xt = jnp.concatenate([df_t, sf_t], axis=2)

    wsp_t = W_sp.T
    wdn_t = W_dn.T
    wc = jnp.transpose(W_ds[:, :, 0, :], (2, 1, 0)).reshape(3 * CDN, CDN)
    inv = jnp.float32(1.0) / jnp.sqrt(jnp.float32(1.0 + 1e-5))
    row = lambda v: v.reshape(1, v.shape[0])

    grid = (bs // BBLK,)
    blk = lambda shape: pl.BlockSpec(shape, lambda i: (i,) + (0,) * (len(shape) - 1))
    full = lambda shape: pl.BlockSpec(shape, lambda i: (0,) * len(shape))

    sp_pre, out_pre, coor_s = pl.pallas_call(
        _body,
        grid=grid,
        in_specs=[blk((BBLK, N_, DLANES + CSP)), blk((BBLK, N_, CO)),
                  full((2 * CSP, CSP)), full((2 * CDN, CDN)), full((3 * CDN, CDN)),
                  full((1, CSP)), full((1, CSP)), full((1, CSP)),
                  full((1, CDN)), full((1, CDN)), full((1, CDN)),
                  full((1, CDN)), full((1, CDN)), full((1, CDN))],
        out_specs=[blk((BBLK, M_, CSP)),
                   blk((BBLK, M_, P_ // 2, CDN)),
                   blk((BBLK, M_, CO))],
        out_shape=[jax.ShapeDtypeStruct((bs, M_, CSP), jnp.float32),
                   jax.ShapeDtypeStruct((bs, M_, P_ // 2, CDN), jnp.float32),
                   jax.ShapeDtypeStruct((bs, M_, CO), jnp.float32)],
        scratch_shapes=[pltpu.VMEM((BBLK, M_, N_), jnp.float32)],
    )(xt, stk_coor, wsp_t, wdn_t, wc,
      row(b_sp), row(g_sp * inv), row(be_sp),
      row(b_dn), row(g_dn * inv), row(be_dn),
      row(b_ds), row(g_ds * inv), row(be_ds))

    sparse_out = jnp.transpose(sp_pre, (0, 2, 1))
    out = jnp.transpose(out_pre, (0, 3, 1, 2))
    return (sparse_out, out, coor_s)
